# baseline (speedup 1.0000x reference)
import functools

import jax
import jax.numpy as jnp
import numpy as np
from jax.experimental import pallas as pl
from jax.experimental.pallas import tpu as pltpu

D = 1024
H = 16
HD = 64
WIN = 512
DFF = 2048
E = 8
S = 2048
CAP = 256
QB = 512
KB = 2 * WIN
TS = 256
NF = 2
NEG = -1e9


def _rot_half(u):
    n = u.shape[-1]
    left = pltpu.roll(u, n - HD // 2, 1)
    right = pltpu.roll(u, HD // 2, 1)
    lane = jax.lax.broadcasted_iota(jnp.int32, u.shape, 1)
    return jnp.where((lane & (HD - 1)) < HD // 2, -left, right)


def _qkv_body(x_ref, w1n_ref, wq_ref, wk_ref, wv_ref, cos_ref,
              sin_ref, q_ref, k_ref, v_ref):
    x = x_ref[...]
    ms = jnp.mean(x * x, axis=-1, keepdims=True)
    xn = x * jax.lax.rsqrt(ms + 1e-6) * w1n_ref[...]
    q = jnp.dot(xn, wq_ref[...], preferred_element_type=jnp.float32)
    k = jnp.dot(xn, wk_ref[...], preferred_element_type=jnp.float32)
    v = jnp.dot(xn, wv_ref[...], preferred_element_type=jnp.float32)
    cos = cos_ref[...]
    sin = sin_ref[...]
    q_ref[...] = (q * cos + _rot_half(q) * sin) * (1.0 / np.sqrt(HD))
    k_ref[...] = k * cos + _rot_half(k) * sin
    v_ref[...] = v


def _attn_body(q_ref, k_ref, v_ref, mask_ref, o_ref):
    sb = pl.program_id(1)

    def head(i, ks, vs, msk):
        sl = slice(i * HD, (i + 1) * HD)
        qh = q_ref[:, sl]
        s = jax.lax.dot_general(
            qh, ks[:, sl], (((1,), (1,)), ((), ())),
            preferred_element_type=jnp.float32) + msk
        p = jnp.exp(s)
        o = jnp.dot(p, vs[:, sl], preferred_element_type=jnp.float32)
        den = jnp.sum(p, axis=-1, keepdims=True)
        o_ref[:, sl] = o / den

    @pl.when(sb == 0)
    def _():
        for i in range(2):
            head(i, k_ref[0:WIN, :], v_ref[0:WIN, :],
                 mask_ref[0, :, 0:WIN])

    @pl.when(sb > 0)
    def _():
        start = (sb - 1) * QB
        for i in range(2):
            head(i, k_ref[pl.ds(start, KB), :], v_ref[pl.ds(start, KB), :],
                 mask_ref[1])


def _proj_router_body(a_ref, wo_ref, x_ref, w2n_ref, gw_ref, ut_ref,
                      h_ref, xn_ref, waT_ref, aux_ref, slot_ref,
                      acc_ref, wacc_ref):
    t = pl.program_id(0)
    h = x_ref[...] + jnp.dot(a_ref[...], wo_ref[...],
                             preferred_element_type=jnp.float32)
    h_ref[...] = h
    ms = jnp.mean(h * h, axis=-1, keepdims=True)
    xn = h * jax.lax.rsqrt(ms + 1e-6) * w2n_ref[...]
    xn_ref[...] = xn.astype(jnp.bfloat16)
    logits = jnp.dot(xn, gw_ref[...], preferred_element_type=jnp.float32)
    mx = jnp.max(logits, axis=-1, keepdims=True)
    ex = jnp.exp(logits - mx)
    gs = ex / jnp.sum(ex, axis=-1, keepdims=True)

    lane = jax.lax.broadcasted_iota(jnp.int32, (TS, E), 1)
    t1 = jnp.max(gs, axis=-1, keepdims=True)
    i1 = jnp.min(jnp.where(gs == t1, lane, E), axis=-1, keepdims=True)
    g2 = jnp.where(lane == i1, -1.0, gs)
    t2 = jnp.max(g2, axis=-1, keepdims=True)
    i2 = jnp.min(jnp.where(g2 == t2, lane, E), axis=-1, keepdims=True)
    assigned = (lane == i1) | (lane == i2)
    wa = jnp.where(assigned, gs, -1.0)
    waT = wa.T
    waT_ref[:, 0, pl.ds(t * TS, TS)] = waT
    wacc_ref[:, pl.ds(t * TS, TS)] = waT

    @pl.when(t == 0)
    def _():
        acc_ref[...] = jnp.zeros_like(acc_ref)
    acc_ref[...] += jnp.sum(gs, axis=0, keepdims=True)

    @pl.when(t == pl.num_programs(0) - 1)
    def _():
        phi = acc_ref[...] / float(S)
        aux_ref[...] = E * jnp.sum(phi * phi, axis=1, keepdims=True)
        _rank(wacc_ref[...], ut_ref, slot_ref)


def _rank(w, ut_ref, slot_ref):
    wi = jax.lax.bitcast_convert_type(w, jnp.int32)

    def cnt_gt(x):
        return jnp.sum((wi > x).astype(jnp.float32), axis=1, keepdims=True)

    def body(_, c):
        lo, hi = c
        mid = jax.lax.shift_right_logical(lo + hi, 1)
        p = cnt_gt(mid) < CAP
        return (jnp.where(p, lo, mid + 1), jnp.where(p, mid, hi))

    lo0 = jnp.zeros((E, 1), jnp.int32)
    hi0 = jnp.full((E, 1), 0x3F800001, jnp.int32)
    _, theta = jax.lax.fori_loop(0, 31, body, (lo0, hi0))
    r = CAP - cnt_gt(theta)
    gt = wi > theta
    eq = wi == theta
    ut = ut_ref[...]
    eqrank = jax.lax.dot_general(
        eq.astype(jnp.bfloat16), ut, (((1,), (0,)), ((), ())),
        preferred_element_type=jnp.float32)
    kept = gt | (eq & (eqrank < r))
    slot_x = jax.lax.dot_general(
        kept.astype(jnp.bfloat16), ut, (((1,), (0,)), ((), ())),
        preferred_element_type=jnp.float32)
    slot_ref[:, 0, :] = jnp.where(kept, slot_x.astype(jnp.int32), -1)


def _expert_body(xn_ref, h_ref, slot_ref, wa_ref, w1f_ref, w2_ref,
                 comb_ref, disp_scr, ws_scr, h12_scr):
    e = pl.program_id(0)
    f = pl.program_id(1)
    DC = D // NF

    @pl.when(jnp.logical_and(e == 0, f == 0))
    def _():
        comb_ref[...] = h_ref[...]

    def onehot():
        slot = slot_ref[0, :, :]
        srow = jax.lax.broadcasted_iota(jnp.int32, (CAP, S), 0)
        return (slot == srow).astype(jnp.bfloat16)

    @pl.when(f == 0)
    def _():
        o = onehot()
        disp_scr[...] = jnp.dot(o, xn_ref[...],
                                preferred_element_type=jnp.float32
                                ).astype(jnp.bfloat16)
        wa = wa_ref[0, :, :]
        ws_scr[...] = jnp.sum(o.astype(jnp.float32) * wa, axis=1,
                              keepdims=True)

    for fs in range(NF):
        @pl.when(f == fs)
        def _():
            dispf = disp_scr[:, fs * DC:(fs + 1) * DC]
            part = jnp.dot(dispf, w1f_ref[0].astype(jnp.bfloat16),
                           preferred_element_type=jnp.float32)
            if fs == 0:
                h12_scr[...] = part
            else:
                h12_scr[...] += part

    @pl.when(f == NF - 1)
    def _():
        h12 = h12_scr[...]
        h1 = h12[:, 0:DFF]
        h2 = h12[:, DFF:2 * DFF]
        act = (h1 * (h2 * jax.nn.sigmoid(h2))).astype(jnp.bfloat16)
        oe = jnp.dot(act, w2_ref[0].astype(jnp.bfloat16),
                     preferred_element_type=jnp.float32)
        oew = (oe * ws_scr[...]).astype(jnp.bfloat16)
        comb_ref[...] += jax.lax.dot_general(
            onehot(), oew, (((0,), (0,)), ((), ())),
            preferred_element_type=jnp.float32)


def _rope_tables():
    inv_freq = 1.0 / (10000.0 ** (np.arange(0, HD, 2, dtype=np.float32) / HD))
    t = np.arange(S, dtype=np.float32)
    freqs = np.einsum('i,j->ij', t, inv_freq)
    emb = np.concatenate([freqs, freqs], axis=-1)
    cos = np.tile(np.cos(emb), (1, H)).astype(np.float32)
    sin = np.tile(np.sin(emb), (1, H)).astype(np.float32)
    p = np.zeros((D, D), dtype=np.float32)
    for h in range(H):
        b = h * HD
        half = HD // 2
        for i in range(half):
            p[b + half + i, b + i] = -1.0
            p[b + i, b + half + i] = 1.0
    return cos, sin, p


def _attn_masks():
    r = np.arange(QB)[:, None]
    c = np.arange(KB)[None, :]
    d0 = r - c
    m0 = np.where((d0 < 0) | (d0 >= WIN), NEG, 0.0)
    d1 = r + QB - c
    m1 = np.where((d1 < 0) | (d1 >= WIN), NEG, 0.0)
    return np.stack([m0, m1]).astype(np.float32)


_COS, _SIN, _ROTP = _rope_tables()
_MASKS = _attn_masks()
_UT = np.triu(np.ones((S, S), dtype=np.float32), 1)


def kernel(x, Wq, Wk, Wv, Wo, norm1_w, norm2_w, gate_w, expert_w1, expert_w2):
    xf = x.reshape(S, D)
    cos = jnp.asarray(_COS)
    sin = jnp.asarray(_SIN)

    full = lambda shape: pl.BlockSpec(shape, lambda *_: (0,) * len(shape))
    rows = lambda bs: pl.BlockSpec((bs, D), lambda t: (t, 0))

    q, k, v = pl.pallas_call(
        _qkv_body,
        grid=(S // TS,),
        in_specs=[rows(TS), full((1, D)), full((D, D)), full((D, D)),
                  full((D, D)), rows(TS), rows(TS)],
        out_specs=[rows(TS), rows(TS), rows(TS)],
        out_shape=[jax.ShapeDtypeStruct((S, D), jnp.float32)] * 3,
    )(xf, norm1_w.reshape(1, D), Wq, Wk, Wv, cos, sin)

    masks = jnp.asarray(_MASKS)
    attn = pl.pallas_call(
        _attn_body,
        grid=(H // 2, S // QB),
        in_specs=[pl.BlockSpec((QB, 2 * HD), lambda hp, sb: (sb, hp)),
                  pl.BlockSpec((S, 2 * HD), lambda hp, sb: (0, hp)),
                  pl.BlockSpec((S, 2 * HD), lambda hp, sb: (0, hp)),
                  pl.BlockSpec((2, QB, KB), lambda hp, sb: (0, 0, 0))],
        out_specs=pl.BlockSpec((QB, 2 * HD), lambda hp, sb: (sb, hp)),
        out_shape=jax.ShapeDtypeStruct((S, D), jnp.float32),
    )(q, k, v, masks)

    ut = jnp.asarray(_UT, dtype=jnp.bfloat16)
    h, xn2, waT, aux, slotT = pl.pallas_call(
        _proj_router_body,
        grid=(S // TS,),
        in_specs=[rows(TS), full((D, D)), rows(TS), full((1, D)),
                  full((D, E)), full((S, S))],
        out_specs=[rows(TS), rows(TS),
                   full((E, 1, S)),
                   full((1, 1)),
                   full((E, 1, S))],
        out_shape=[jax.ShapeDtypeStruct((S, D), jnp.float32),
                   jax.ShapeDtypeStruct((S, D), jnp.bfloat16),
                   jax.ShapeDtypeStruct((E, 1, S), jnp.float32),
                   jax.ShapeDtypeStruct((1, 1), jnp.float32),
                   jax.ShapeDtypeStruct((E, 1, S), jnp.int32)],
        scratch_shapes=[pltpu.VMEM((1, E), jnp.float32),
                        pltpu.VMEM((E, S), jnp.float32)],
    )(attn, Wo, xf, norm2_w.reshape(1, D), gate_w, ut)

    y = pl.pallas_call(
        _expert_body,
        grid=(E, NF),
        in_specs=[full((S, D)), full((S, D)),
                  pl.BlockSpec((1, 1, S), lambda e, f: (e, 0, 0)),
                  pl.BlockSpec((1, 1, S), lambda e, f: (e, 0, 0)),
                  pl.BlockSpec((1, D // NF, 2 * DFF), lambda e, f: (e, f, 0)),
                  pl.BlockSpec((1, DFF, D), lambda e, f: (e, 0, 0))],
        out_specs=full((S, D)),
        out_shape=jax.ShapeDtypeStruct((S, D), jnp.float32),
        scratch_shapes=[pltpu.VMEM((CAP, D), jnp.bfloat16),
                        pltpu.VMEM((CAP, 1), jnp.float32),
                        pltpu.VMEM((CAP, 2 * DFF), jnp.float32)],
    )(xn2, h, slotT, waT, expert_w1, expert_w2)

    return y.reshape(1, S, D), aux.reshape(())


# speedup vs baseline: 1.0386x; 1.0386x over previous
import functools

import jax
import jax.numpy as jnp
import numpy as np
from jax.experimental import pallas as pl
from jax.experimental.pallas import tpu as pltpu

D = 1024
H = 16
HD = 64
WIN = 512
DFF = 2048
E = 8
S = 2048
CAP = 256
QB = 512
KB = 2 * WIN
TS = 256
NF = 2
NEG = -1e9


def _rot_half(u):
    n = u.shape[-1]
    left = pltpu.roll(u, n - HD // 2, 1)
    right = pltpu.roll(u, HD // 2, 1)
    lane = jax.lax.broadcasted_iota(jnp.int32, u.shape, 1)
    return jnp.where((lane & (HD - 1)) < HD // 2, -left, right)


def _qkv_body(x_ref, w1n_ref, wq_ref, wk_ref, wv_ref, cos_ref,
              sin_ref, q_ref, k_ref, v_ref):
    x = x_ref[...]
    ms = jnp.mean(x * x, axis=-1, keepdims=True)
    xn = x * jax.lax.rsqrt(ms + 1e-6) * w1n_ref[...]
    q = jnp.dot(xn, wq_ref[...], preferred_element_type=jnp.float32)
    k = jnp.dot(xn, wk_ref[...], preferred_element_type=jnp.float32)
    v = jnp.dot(xn, wv_ref[...], preferred_element_type=jnp.float32)
    cos = cos_ref[...]
    sin = sin_ref[...]
    q_ref[...] = (q * cos + _rot_half(q) * sin) * (1.0 / np.sqrt(HD))
    k_ref[...] = k * cos + _rot_half(k) * sin
    v_ref[...] = v


def _attn_body(q_ref, k_ref, v_ref, mask_ref, o_ref):
    sb = pl.program_id(1)

    def head(i, ks, vs, msk):
        sl = slice(i * HD, (i + 1) * HD)
        qh = q_ref[:, sl]
        s = jax.lax.dot_general(
            qh, ks[:, sl], (((1,), (1,)), ((), ())),
            preferred_element_type=jnp.float32) + msk
        p = jnp.exp(s)
        o = jnp.dot(p, vs[:, sl], preferred_element_type=jnp.float32)
        den = jnp.sum(p, axis=-1, keepdims=True)
        o_ref[:, sl] = o / den

    @pl.when(sb == 0)
    def _():
        for i in range(2):
            head(i, k_ref[0:WIN, :], v_ref[0:WIN, :],
                 mask_ref[0, :, 0:WIN])

    @pl.when(sb > 0)
    def _():
        start = (sb - 1) * QB
        for i in range(2):
            head(i, k_ref[pl.ds(start, KB), :], v_ref[pl.ds(start, KB), :],
                 mask_ref[1])


def _proj_router_body(a_ref, wo_ref, x_ref, w2n_ref, gw_ref, ut_ref,
                      h_ref, xn_ref, waT_ref, aux_ref, slot_ref,
                      acc_ref, wacc_ref):
    t = pl.program_id(0)
    h = x_ref[...] + jnp.dot(a_ref[...], wo_ref[...],
                             preferred_element_type=jnp.float32)
    h_ref[...] = h
    ms = jnp.mean(h * h, axis=-1, keepdims=True)
    xn = h * jax.lax.rsqrt(ms + 1e-6) * w2n_ref[...]
    xn_ref[...] = xn.astype(jnp.bfloat16)
    logits = jnp.dot(xn, gw_ref[...], preferred_element_type=jnp.float32)
    mx = jnp.max(logits, axis=-1, keepdims=True)
    ex = jnp.exp(logits - mx)
    gs = ex / jnp.sum(ex, axis=-1, keepdims=True)

    lane = jax.lax.broadcasted_iota(jnp.int32, (TS, E), 1)
    t1 = jnp.max(gs, axis=-1, keepdims=True)
    i1 = jnp.min(jnp.where(gs == t1, lane, E), axis=-1, keepdims=True)
    g2 = jnp.where(lane == i1, -1.0, gs)
    t2 = jnp.max(g2, axis=-1, keepdims=True)
    i2 = jnp.min(jnp.where(g2 == t2, lane, E), axis=-1, keepdims=True)
    assigned = (lane == i1) | (lane == i2)
    wa = jnp.where(assigned, gs, -1.0)
    waT = wa.T
    waT_ref[:, 0, pl.ds(t * TS, TS)] = waT
    wacc_ref[:, pl.ds(t * TS, TS)] = waT

    @pl.when(t == 0)
    def _():
        acc_ref[...] = jnp.zeros_like(acc_ref)
    acc_ref[...] += jnp.sum(gs, axis=0, keepdims=True)

    @pl.when(t == pl.num_programs(0) - 1)
    def _():
        phi = acc_ref[...] / float(S)
        aux_ref[...] = E * jnp.sum(phi * phi, axis=1, keepdims=True)
        _rank(wacc_ref[...], ut_ref, slot_ref)


def _rank(w, ut_ref, slot_ref):
    wi = jax.lax.bitcast_convert_type(w, jnp.int32)

    def cnt_gt(x):
        return jnp.sum((wi > x).astype(jnp.float32), axis=1, keepdims=True)

    def body(_, c):
        lo, hi = c
        mid = jax.lax.shift_right_logical(lo + hi, 1)
        p = cnt_gt(mid) < CAP
        return (jnp.where(p, lo, mid + 1), jnp.where(p, mid, hi))

    lo0 = jnp.zeros((E, 1), jnp.int32)
    hi0 = jnp.full((E, 1), 0x3F800001, jnp.int32)
    _, theta = jax.lax.fori_loop(0, 31, body, (lo0, hi0))
    r = CAP - cnt_gt(theta)
    gt = wi > theta
    eq = wi == theta
    ut = ut_ref[...]
    eqrank = jax.lax.dot_general(
        eq.astype(jnp.bfloat16), ut, (((1,), (0,)), ((), ())),
        preferred_element_type=jnp.float32)
    kept = gt | (eq & (eqrank < r))
    slot_x = jax.lax.dot_general(
        kept.astype(jnp.bfloat16), ut, (((1,), (0,)), ((), ())),
        preferred_element_type=jnp.float32)
    slot_ref[:, 0, :] = jnp.where(kept, slot_x.astype(jnp.int32), -1)


def _expert_body(xn_ref, h_ref, slot_ref, wa_ref, w1a_ref, w1b_ref, w2f_ref,
                 comb_ref, o_scr, disp_scr, ws_scr, oe_scr):
    e = pl.program_id(0)
    f = pl.program_id(1)

    @pl.when(jnp.logical_and(e == 0, f == 0))
    def _():
        comb_ref[...] = h_ref[...]

    @pl.when(f == 0)
    def _():
        slot = slot_ref[0, :, :]
        srow = jax.lax.broadcasted_iota(jnp.int32, (CAP, S), 0)
        o = (slot == srow).astype(jnp.bfloat16)
        o_scr[...] = o
        disp_scr[...] = jnp.dot(o, xn_ref[...],
                                preferred_element_type=jnp.float32
                                ).astype(jnp.bfloat16)
        wa = wa_ref[0, :, :]
        ws_scr[...] = jnp.sum(o.astype(jnp.float32) * wa, axis=1,
                              keepdims=True)

    disp = disp_scr[...]
    h1 = jnp.dot(disp, w1a_ref[0].astype(jnp.bfloat16),
                 preferred_element_type=jnp.float32)
    h2 = jnp.dot(disp, w1b_ref[0].astype(jnp.bfloat16),
                 preferred_element_type=jnp.float32)
    act = (h1 * (h2 * jax.nn.sigmoid(h2))).astype(jnp.bfloat16)
    oe = jnp.dot(act, w2f_ref[0].astype(jnp.bfloat16),
                 preferred_element_type=jnp.float32)

    @pl.when(f == 0)
    def _():
        oe_scr[...] = oe

    @pl.when(f > 0)
    def _():
        oe_scr[...] += oe

    @pl.when(f == NF - 1)
    def _():
        oew = (oe_scr[...] * ws_scr[...]).astype(jnp.bfloat16)
        comb_ref[...] += jax.lax.dot_general(
            o_scr[...], oew, (((0,), (0,)), ((), ())),
            preferred_element_type=jnp.float32)


def _rope_tables():
    inv_freq = 1.0 / (10000.0 ** (np.arange(0, HD, 2, dtype=np.float32) / HD))
    t = np.arange(S, dtype=np.float32)
    freqs = np.einsum('i,j->ij', t, inv_freq)
    emb = np.concatenate([freqs, freqs], axis=-1)
    cos = np.tile(np.cos(emb), (1, H)).astype(np.float32)
    sin = np.tile(np.sin(emb), (1, H)).astype(np.float32)
    p = np.zeros((D, D), dtype=np.float32)
    for h in range(H):
        b = h * HD
        half = HD // 2
        for i in range(half):
            p[b + half + i, b + i] = -1.0
            p[b + i, b + half + i] = 1.0
    return cos, sin, p


def _attn_masks():
    r = np.arange(QB)[:, None]
    c = np.arange(KB)[None, :]
    d0 = r - c
    m0 = np.where((d0 < 0) | (d0 >= WIN), NEG, 0.0)
    d1 = r + QB - c
    m1 = np.where((d1 < 0) | (d1 >= WIN), NEG, 0.0)
    return np.stack([m0, m1]).astype(np.float32)


_COS, _SIN, _ROTP = _rope_tables()
_MASKS = _attn_masks()
_UT = np.triu(np.ones((S, S), dtype=np.float32), 1)


def kernel(x, Wq, Wk, Wv, Wo, norm1_w, norm2_w, gate_w, expert_w1, expert_w2):
    xf = x.reshape(S, D)
    cos = jnp.asarray(_COS)
    sin = jnp.asarray(_SIN)

    full = lambda shape: pl.BlockSpec(shape, lambda *_: (0,) * len(shape))
    rows = lambda bs: pl.BlockSpec((bs, D), lambda t: (t, 0))

    q, k, v = pl.pallas_call(
        _qkv_body,
        grid=(S // TS,),
        in_specs=[rows(TS), full((1, D)), full((D, D)), full((D, D)),
                  full((D, D)), rows(TS), rows(TS)],
        out_specs=[rows(TS), rows(TS), rows(TS)],
        out_shape=[jax.ShapeDtypeStruct((S, D), jnp.float32)] * 3,
    )(xf, norm1_w.reshape(1, D), Wq, Wk, Wv, cos, sin)

    masks = jnp.asarray(_MASKS)
    attn = pl.pallas_call(
        _attn_body,
        grid=(H // 2, S // QB),
        in_specs=[pl.BlockSpec((QB, 2 * HD), lambda hp, sb: (sb, hp)),
                  pl.BlockSpec((S, 2 * HD), lambda hp, sb: (0, hp)),
                  pl.BlockSpec((S, 2 * HD), lambda hp, sb: (0, hp)),
                  pl.BlockSpec((2, QB, KB), lambda hp, sb: (0, 0, 0))],
        out_specs=pl.BlockSpec((QB, 2 * HD), lambda hp, sb: (sb, hp)),
        out_shape=jax.ShapeDtypeStruct((S, D), jnp.float32),
    )(q, k, v, masks)

    ut = jnp.asarray(_UT, dtype=jnp.bfloat16)
    h, xn2, waT, aux, slotT = pl.pallas_call(
        _proj_router_body,
        grid=(S // TS,),
        in_specs=[rows(TS), full((D, D)), rows(TS), full((1, D)),
                  full((D, E)), full((S, S))],
        out_specs=[rows(TS), rows(TS),
                   full((E, 1, S)),
                   full((1, 1)),
                   full((E, 1, S))],
        out_shape=[jax.ShapeDtypeStruct((S, D), jnp.float32),
                   jax.ShapeDtypeStruct((S, D), jnp.bfloat16),
                   jax.ShapeDtypeStruct((E, 1, S), jnp.float32),
                   jax.ShapeDtypeStruct((1, 1), jnp.float32),
                   jax.ShapeDtypeStruct((E, 1, S), jnp.int32)],
        scratch_shapes=[pltpu.VMEM((1, E), jnp.float32),
                        pltpu.VMEM((E, S), jnp.float32)],
    )(attn, Wo, xf, norm2_w.reshape(1, D), gate_w, ut)

    y = pl.pallas_call(
        _expert_body,
        grid=(E, NF),
        in_specs=[full((S, D)), full((S, D)),
                  pl.BlockSpec((1, 1, S), lambda e, f: (e, 0, 0)),
                  pl.BlockSpec((1, 1, S), lambda e, f: (e, 0, 0)),
                  pl.BlockSpec((1, D, DFF // NF), lambda e, f: (e, 0, f)),
                  pl.BlockSpec((1, D, DFF // NF), lambda e, f: (e, 0, NF + f)),
                  pl.BlockSpec((1, DFF // NF, D), lambda e, f: (e, f, 0))],
        out_specs=full((S, D)),
        out_shape=jax.ShapeDtypeStruct((S, D), jnp.float32),
        scratch_shapes=[pltpu.VMEM((CAP, S), jnp.bfloat16),
                        pltpu.VMEM((CAP, D), jnp.bfloat16),
                        pltpu.VMEM((CAP, 1), jnp.float32),
                        pltpu.VMEM((CAP, D), jnp.float32)],
    )(xn2, h, slotT, waT, expert_w1, expert_w1, expert_w2)

    return y.reshape(1, S, D), aux.reshape(())


# speedup vs baseline: 1.0660x; 1.0264x over previous
import functools

import jax
import jax.numpy as jnp
import numpy as np
from jax.experimental import pallas as pl
from jax.experimental.pallas import tpu as pltpu

D = 1024
H = 16
HD = 64
WIN = 512
DFF = 2048
E = 8
S = 2048
CAP = 256
QB = 512
KB = 2 * WIN
TS = 512
NF = 2
NEG = -1e9


def _rot_half(u):
    n = u.shape[-1]
    left = pltpu.roll(u, n - HD // 2, 1)
    right = pltpu.roll(u, HD // 2, 1)
    lane = jax.lax.broadcasted_iota(jnp.int32, u.shape, 1)
    return jnp.where((lane & (HD - 1)) < HD // 2, -left, right)


def _qkv_body(x_ref, w1n_ref, wq_ref, wk_ref, wv_ref, cos_ref,
              sin_ref, q_ref, k_ref, v_ref):
    x = x_ref[...]
    ms = jnp.mean(x * x, axis=-1, keepdims=True)
    xn = x * jax.lax.rsqrt(ms + 1e-6) * w1n_ref[...]
    q = jnp.dot(xn, wq_ref[...], preferred_element_type=jnp.float32)
    k = jnp.dot(xn, wk_ref[...], preferred_element_type=jnp.float32)
    v = jnp.dot(xn, wv_ref[...], preferred_element_type=jnp.float32)
    cos = cos_ref[...]
    sin = sin_ref[...]
    q_ref[...] = (q * cos + _rot_half(q) * sin) * (1.0 / np.sqrt(HD))
    k_ref[...] = k * cos + _rot_half(k) * sin
    v_ref[...] = v


def _attn_body(q_ref, k_ref, v_ref, mask_ref, o_ref):
    sb = pl.program_id(1)

    def head(i, ks, vs, msk):
        sl = slice(i * HD, (i + 1) * HD)
        qh = q_ref[:, sl]
        s = jax.lax.dot_general(
            qh, ks[:, sl], (((1,), (1,)), ((), ())),
            preferred_element_type=jnp.float32) + msk
        p = jnp.exp(s)
        o = jnp.dot(p, vs[:, sl], preferred_element_type=jnp.float32)
        den = jnp.sum(p, axis=-1, keepdims=True)
        o_ref[:, sl] = o / den

    @pl.when(sb == 0)
    def _():
        for i in range(2):
            head(i, k_ref[0:WIN, :], v_ref[0:WIN, :],
                 mask_ref[0, :, 0:WIN])

    @pl.when(sb > 0)
    def _():
        start = (sb - 1) * QB
        for i in range(2):
            head(i, k_ref[pl.ds(start, KB), :], v_ref[pl.ds(start, KB), :],
                 mask_ref[1])


def _proj_router_body(a_ref, wo_ref, x_ref, w2n_ref, gw_ref, ut_ref,
                      h_ref, xn_ref, waT_ref, aux_ref, slot_ref,
                      acc_ref, wacc_ref):
    t = pl.program_id(0)
    h = x_ref[...] + jnp.dot(a_ref[...], wo_ref[...],
                             preferred_element_type=jnp.float32)
    h_ref[...] = h
    ms = jnp.mean(h * h, axis=-1, keepdims=True)
    xn = h * jax.lax.rsqrt(ms + 1e-6) * w2n_ref[...]
    xn_ref[...] = xn.astype(jnp.bfloat16)
    logits = jnp.dot(xn, gw_ref[...], preferred_element_type=jnp.float32)
    mx = jnp.max(logits, axis=-1, keepdims=True)
    ex = jnp.exp(logits - mx)
    gs = ex / jnp.sum(ex, axis=-1, keepdims=True)

    lane = jax.lax.broadcasted_iota(jnp.int32, (TS, E), 1)
    t1 = jnp.max(gs, axis=-1, keepdims=True)
    i1 = jnp.min(jnp.where(gs == t1, lane, E), axis=-1, keepdims=True)
    g2 = jnp.where(lane == i1, -1.0, gs)
    t2 = jnp.max(g2, axis=-1, keepdims=True)
    i2 = jnp.min(jnp.where(g2 == t2, lane, E), axis=-1, keepdims=True)
    assigned = (lane == i1) | (lane == i2)
    wa = jnp.where(assigned, gs, -1.0)
    waT = wa.T
    waT_ref[:, 0, pl.ds(t * TS, TS)] = waT
    wacc_ref[:, pl.ds(t * TS, TS)] = waT

    @pl.when(t == 0)
    def _():
        acc_ref[...] = jnp.zeros_like(acc_ref)
    acc_ref[...] += jnp.sum(gs, axis=0, keepdims=True)

    @pl.when(t == pl.num_programs(0) - 1)
    def _():
        phi = acc_ref[...] / float(S)
        aux_ref[...] = E * jnp.sum(phi * phi, axis=1, keepdims=True)
        _rank(wacc_ref[...], ut_ref, slot_ref)


def _rank(w, ut_ref, slot_ref):
    wi = jax.lax.bitcast_convert_type(w, jnp.int32)

    def cnt_gt(x):
        return jnp.sum((wi > x).astype(jnp.float32), axis=1, keepdims=True)

    def body(_, c):
        lo, hi = c
        mid = jax.lax.shift_right_logical(lo + hi, 1)
        p = cnt_gt(mid) < CAP
        return (jnp.where(p, lo, mid + 1), jnp.where(p, mid, hi))

    lo0 = jnp.zeros((E, 1), jnp.int32)
    hi0 = jnp.full((E, 1), 0x3F800001, jnp.int32)
    _, theta = jax.lax.fori_loop(0, 31, body, (lo0, hi0))
    r = CAP - cnt_gt(theta)
    gt = wi > theta
    eq = wi == theta
    ut = ut_ref[...]
    eqrank = jax.lax.dot_general(
        eq.astype(jnp.bfloat16), ut, (((1,), (0,)), ((), ())),
        preferred_element_type=jnp.float32)
    kept = gt | (eq & (eqrank < r))
    slot_x = jax.lax.dot_general(
        kept.astype(jnp.bfloat16), ut, (((1,), (0,)), ((), ())),
        preferred_element_type=jnp.float32)
    slot_ref[:, 0, :] = jnp.where(kept, slot_x.astype(jnp.int32), -1)


def _expert_body(xn_ref, h_ref, slot_ref, wa_ref, w1a_ref, w1b_ref, w2f_ref,
                 comb_ref, o_scr, disp_scr, ws_scr, oe_scr):
    e = pl.program_id(0)
    f = pl.program_id(1)

    @pl.when(jnp.logical_and(e == 0, f == 0))
    def _():
        comb_ref[...] = h_ref[...]

    @pl.when(f == 0)
    def _():
        slot = slot_ref[0, :, :]
        srow = jax.lax.broadcasted_iota(jnp.int32, (CAP, S), 0)
        o = (slot == srow).astype(jnp.bfloat16)
        o_scr[...] = o
        disp_scr[...] = jnp.dot(o, xn_ref[...],
                                preferred_element_type=jnp.float32
                                ).astype(jnp.bfloat16)
        wa = wa_ref[0, :, :]
        ws_scr[...] = jnp.sum(o.astype(jnp.float32) * wa, axis=1,
                              keepdims=True)

    disp = disp_scr[...]
    h1 = jnp.dot(disp, w1a_ref[0].astype(jnp.bfloat16),
                 preferred_element_type=jnp.float32)
    h2 = jnp.dot(disp, w1b_ref[0].astype(jnp.bfloat16),
                 preferred_element_type=jnp.float32)
    act = (h1 * (h2 * jax.nn.sigmoid(h2))).astype(jnp.bfloat16)
    oe = jnp.dot(act, w2f_ref[0].astype(jnp.bfloat16),
                 preferred_element_type=jnp.float32)

    @pl.when(f == 0)
    def _():
        oe_scr[...] = oe

    @pl.when(f > 0)
    def _():
        oe_scr[...] += oe

    @pl.when(f == NF - 1)
    def _():
        oew = (oe_scr[...] * ws_scr[...]).astype(jnp.bfloat16)
        comb_ref[...] += jax.lax.dot_general(
            o_scr[...], oew, (((0,), (0,)), ((), ())),
            preferred_element_type=jnp.float32)


def _rope_tables():
    inv_freq = 1.0 / (10000.0 ** (np.arange(0, HD, 2, dtype=np.float32) / HD))
    t = np.arange(S, dtype=np.float32)
    freqs = np.einsum('i,j->ij', t, inv_freq)
    emb = np.concatenate([freqs, freqs], axis=-1)
    cos = np.tile(np.cos(emb), (1, H)).astype(np.float32)
    sin = np.tile(np.sin(emb), (1, H)).astype(np.float32)
    return cos, sin


def _attn_masks():
    r = np.arange(QB)[:, None]
    c = np.arange(KB)[None, :]
    d0 = r - c
    m0 = np.where((d0 < 0) | (d0 >= WIN), NEG, 0.0)
    d1 = r + QB - c
    m1 = np.where((d1 < 0) | (d1 >= WIN), NEG, 0.0)
    return np.stack([m0, m1]).astype(np.float32)


_COS, _SIN = _rope_tables()
_MASKS = _attn_masks()
_UT = np.triu(np.ones((S, S), dtype=np.float32), 1)


def kernel(x, Wq, Wk, Wv, Wo, norm1_w, norm2_w, gate_w, expert_w1, expert_w2):
    xf = x.reshape(S, D)
    cos = jnp.asarray(_COS)
    sin = jnp.asarray(_SIN)

    full = lambda shape: pl.BlockSpec(shape, lambda *_: (0,) * len(shape))
    rows = lambda bs: pl.BlockSpec((bs, D), lambda t: (t, 0))

    q, k, v = pl.pallas_call(
        _qkv_body,
        grid=(S // TS,),
        in_specs=[rows(TS), full((1, D)), full((D, D)), full((D, D)),
                  full((D, D)), rows(TS), rows(TS)],
        out_specs=[rows(TS), rows(TS), rows(TS)],
        out_shape=[jax.ShapeDtypeStruct((S, D), jnp.float32)] * 3,
    )(xf, norm1_w.reshape(1, D), Wq, Wk, Wv, cos, sin)

    masks = jnp.asarray(_MASKS)
    attn = pl.pallas_call(
        _attn_body,
        grid=(H // 2, S // QB),
        in_specs=[pl.BlockSpec((QB, 2 * HD), lambda hp, sb: (sb, hp)),
                  pl.BlockSpec((S, 2 * HD), lambda hp, sb: (0, hp)),
                  pl.BlockSpec((S, 2 * HD), lambda hp, sb: (0, hp)),
                  pl.BlockSpec((2, QB, KB), lambda hp, sb: (0, 0, 0))],
        out_specs=pl.BlockSpec((QB, 2 * HD), lambda hp, sb: (sb, hp)),
        out_shape=jax.ShapeDtypeStruct((S, D), jnp.float32),
    )(q, k, v, masks)

    ut = jnp.asarray(_UT, dtype=jnp.bfloat16)
    h, xn2, waT, aux, slotT = pl.pallas_call(
        _proj_router_body,
        grid=(S // TS,),
        in_specs=[rows(TS), full((D, D)), rows(TS), full((1, D)),
                  full((D, E)), full((S, S))],
        out_specs=[rows(TS), rows(TS),
                   full((E, 1, S)),
                   full((1, 1)),
                   full((E, 1, S))],
        out_shape=[jax.ShapeDtypeStruct((S, D), jnp.float32),
                   jax.ShapeDtypeStruct((S, D), jnp.bfloat16),
                   jax.ShapeDtypeStruct((E, 1, S), jnp.float32),
                   jax.ShapeDtypeStruct((1, 1), jnp.float32),
                   jax.ShapeDtypeStruct((E, 1, S), jnp.int32)],
        scratch_shapes=[pltpu.VMEM((1, E), jnp.float32),
                        pltpu.VMEM((E, S), jnp.float32)],
    )(attn, Wo, xf, norm2_w.reshape(1, D), gate_w, ut)

    y = pl.pallas_call(
        _expert_body,
        grid=(E, NF),
        in_specs=[full((S, D)), full((S, D)),
                  pl.BlockSpec((1, 1, S), lambda e, f: (e, 0, 0)),
                  pl.BlockSpec((1, 1, S), lambda e, f: (e, 0, 0)),
                  pl.BlockSpec((1, D, DFF // NF), lambda e, f: (e, 0, f)),
                  pl.BlockSpec((1, D, DFF // NF), lambda e, f: (e, 0, NF + f)),
                  pl.BlockSpec((1, DFF // NF, D), lambda e, f: (e, f, 0))],
        out_specs=full((S, D)),
        out_shape=jax.ShapeDtypeStruct((S, D), jnp.float32),
        scratch_shapes=[pltpu.VMEM((CAP, S), jnp.bfloat16),
                        pltpu.VMEM((CAP, D), jnp.bfloat16),
                        pltpu.VMEM((CAP, 1), jnp.float32),
                        pltpu.VMEM((CAP, D), jnp.float32)],
    )(xn2, h, slotT, waT, expert_w1, expert_w1, expert_w2)

    return y.reshape(1, S, D), aux.reshape(())
